# baseline (speedup 1.0000x reference)
import functools

import jax
import jax.numpy as jnp
from jax import lax
from jax.experimental import pallas as pl
from jax.experimental.pallas import tpu as pltpu
from jax.experimental.pallas import tpu_sc as plsc

N = 10000
E = 320000
D_IN = 128
D_HID = 128
D_OUT = 64
ATTRACT = 32

NC = 2
NS = 16
NT = NC * NS
CH = 128
EPT = E // NT
NCHUNK = (EPT + CH - 1) // CH
EPT_PAD = NCHUNK * CH
NPAD = 10240
RPT = NPAD // NS
BLK = 640
GRID = NPAD // BLK

_mesh = plsc.VectorSubcoreMesh(core_axis_name="c", subcore_axis_name="s")
f32 = jnp.float32
i32 = jnp.int32


@functools.partial(
    pl.kernel,
    out_type=jax.ShapeDtypeStruct((NC, NPAD), f32),
    mesh=_mesh,
    scratch_types=[
        pltpu.VMEM((NCHUNK, CH), i32),
        pltpu.VMEM((CH,), f32),
        pltpu.VMEM((RPT,), f32),
        pltpu.VMEM_SHARED((NPAD,), f32),
    ],
)
def _sc_hist(dst_hbm, out_hbm, dst_v, ones_v, zb_v, acc_sh):
    cid = lax.axis_index("c")
    sid = lax.axis_index("s")
    wid = cid * NS + sid
    pltpu.sync_copy(dst_hbm.at[wid], dst_v)
    for i in range(CH // 16):
        ones_v[pl.ds(i * 16, 16)] = jnp.ones((16,), f32)
    for i in range(RPT // 16):
        zb_v[pl.ds(i * 16, 16)] = jnp.zeros((16,), f32)
    pltpu.sync_copy(zb_v, acc_sh.at[pl.ds(sid * RPT, RPT)])
    plsc.subcore_barrier()

    def body(j, c):
        pltpu.sync_copy(ones_v, acc_sh.at[dst_v.at[j]], add=True)
        return c

    lax.fori_loop(0, NCHUNK, body, 0)
    plsc.subcore_barrier()
    pltpu.sync_copy(acc_sh.at[pl.ds(sid * RPT, RPT)],
                    out_hbm.at[cid, pl.ds(sid * RPT, RPT)])


D = D_OUT


@functools.partial(
    pl.kernel,
    out_type=jax.ShapeDtypeStruct((NC, NPAD, D), f32),
    mesh=_mesh,
    compiler_params=pltpu.CompilerParams(use_tc_tiling_on_sc=False),
    scratch_types=[
        pltpu.VMEM((NCHUNK, CH), i32),
        pltpu.VMEM((NCHUNK, CH), i32),
        pltpu.VMEM((CH, D), f32),
        pltpu.VMEM((CH, D), f32),
        pltpu.VMEM_SHARED((NPAD, D), f32),
        pltpu.VMEM_SHARED((NPAD, D), f32),
        pltpu.SemaphoreType.DMA,
        pltpu.SemaphoreType.DMA,
    ],
)
def _sc_scatter64(u_hbm, src_hbm, dst_hbm, zeros_hbm, out_hbm,
                  src_v, dst_v, r0, r1, acc_sh, u_sh, sem0, sem1):
    cid = lax.axis_index("c")
    sid = lax.axis_index("s")
    wid = cid * NS + sid
    pltpu.sync_copy(src_hbm.at[wid], src_v)
    pltpu.sync_copy(dst_hbm.at[wid], dst_v)
    pltpu.sync_copy(zeros_hbm, acc_sh.at[pl.ds(sid * RPT, RPT)])
    pltpu.sync_copy(u_hbm.at[pl.ds(sid * RPT, RPT)],
                    u_sh.at[pl.ds(sid * RPT, RPT)])
    plsc.subcore_barrier()

    def fire(j, buf, sem):
        pltpu.async_copy(u_sh.at[src_v.at[j]], buf, sem)

    def drain(j, buf, sem):
        pltpu.make_async_copy(u_sh.at[src_v.at[j]], buf, sem).wait()

    def scat(j, buf):
        pltpu.sync_copy(buf, acc_sh.at[dst_v.at[j]], add=True)

    fire(0, r0, sem0)

    def pair(k, c):
        j0 = 2 * k
        j1 = j0 + 1
        drain(j0, r0, sem0)
        fire(j1, r1, sem1)
        scat(j0, r0)
        drain(j1, r1, sem1)
        fire(j0 + 2, r0, sem0)
        scat(j1, r1)
        return c

    lax.fori_loop(0, (NCHUNK - 1) // 2, pair, 0)
    drain(NCHUNK - 1, r0, sem0)
    scat(NCHUNK - 1, r0)
    plsc.subcore_barrier()
    pltpu.sync_copy(acc_sh.at[pl.ds(sid * RPT, RPT)],
                    out_hbm.at[cid, pl.ds(sid * RPT, RPT)])


@functools.partial(
    pl.kernel,
    out_type=jax.ShapeDtypeStruct((NT, NCHUNK, CH), f32),
    mesh=_mesh,
    compiler_params=pltpu.CompilerParams(use_tc_tiling_on_sc=False,
                                         needs_layout_passes=False),
    scratch_types=[
        pltpu.VMEM((NCHUNK, CH), i32),
        pltpu.VMEM((NCHUNK, CH), i32),
        pltpu.VMEM((CH, D_OUT), f32),
        pltpu.VMEM((CH, D_OUT), f32),
        pltpu.VMEM((CH, D_OUT), f32),
        pltpu.VMEM((CH, D_OUT), f32),
        pltpu.VMEM((NCHUNK, CH), f32),
        pltpu.VMEM((16, 17), f32),
        pltpu.VMEM((16, 17), f32),
        pltpu.VMEM_SHARED((NPAD, D_OUT), f32),
        pltpu.SemaphoreType.DMA,
        pltpu.SemaphoreType.DMA,
    ],
)
def _sc_score(z_hbm, s_hbm, d_hbm, out_hbm,
              s_v, d_v, a0, b0, a1, b1, sc_v, tbuf0, tbuf1, z_sh, sem0, sem1):
    cid = lax.axis_index("c")
    sid = lax.axis_index("s")
    wid = cid * NS + sid
    pltpu.sync_copy(s_hbm.at[wid], s_v)
    pltpu.sync_copy(d_hbm.at[wid], d_v)
    pltpu.sync_copy(z_hbm.at[pl.ds(sid * RPT, RPT)],
                    z_sh.at[pl.ds(sid * RPT, RPT)])
    plsc.subcore_barrier()

    def fire(j, av, bv, sem):
        pltpu.async_copy(z_sh.at[s_v.at[j]], av, sem)
        pltpu.async_copy(z_sh.at[d_v.at[j]], bv, sem)

    def drain(j, av, bv, sem):
        pltpu.make_async_copy(z_sh.at[s_v.at[j]], av, sem).wait()
        pltpu.make_async_copy(z_sh.at[d_v.at[j]], bv, sem).wait()

    lane = lax.iota(i32, 16)

    def compute(j, av, bv):
        def phase1(g, tbuf):
            for t in range(16):
                e = g * 16 + t
                ra = (av[e, pl.ds(0, 16)] * bv[e, pl.ds(0, 16)]
                      + av[e, pl.ds(16, 16)] * bv[e, pl.ds(16, 16)])
                rb = (av[e, pl.ds(32, 16)] * bv[e, pl.ds(32, 16)]
                      + av[e, pl.ds(48, 16)] * bv[e, pl.ds(48, 16)])
                tbuf[t, pl.ds(0, 16)] = ra - rb

        def phase2(g, tbuf):
            acc0 = jnp.zeros((16,), f32)
            acc1 = jnp.zeros((16,), f32)
            for c in range(0, 16, 2):
                acc0 = acc0 + plsc.load_gather(tbuf, [lane, jnp.full((16,), c, i32)])
                acc1 = acc1 + plsc.load_gather(tbuf, [lane, jnp.full((16,), c + 1, i32)])
            sc_v[j, pl.ds(pl.multiple_of(g * 16, 16), 16)] = acc0 + acc1

        for g in range(CH // 16):
            phase1(g, tbuf0)
            phase2(g, tbuf0)

    fire(0, a0, b0, sem0)

    def pair(k, c):
        j0 = 2 * k
        j1 = j0 + 1
        drain(j0, a0, b0, sem0)
        fire(j1, a1, b1, sem1)
        compute(j0, a0, b0)
        drain(j1, a1, b1, sem1)
        fire(j0 + 2, a0, b0, sem0)
        compute(j1, a1, b1)
        return c

    lax.fori_loop(0, (NCHUNK - 1) // 2, pair, 0)
    drain(NCHUNK - 1, a0, b0, sem0)
    compute(NCHUNK - 1, a0, b0)
    pltpu.sync_copy(sc_v, out_hbm.at[wid])


def _mm1_body(degp_ref, x_ref, w1_ref, u1a_ref, u1b_ref, dinv_ref):
    deg = degp_ref[0] + degp_ref[1] + 1.0
    dv = lax.rsqrt(deg)
    dinv_ref[...] = dv
    xw = jnp.dot(x_ref[...], w1_ref[...], preferred_element_type=f32)
    u1a_ref[...] = xw[:, :D_OUT] * dv
    u1b_ref[...] = xw[:, D_OUT:] * dv


_mm1 = pl.pallas_call(
    _mm1_body,
    grid=(GRID,),
    in_specs=[
        pl.BlockSpec((NC, BLK, 1), lambda i: (0, i, 0)),
        pl.BlockSpec((BLK, D_IN), lambda i: (i, 0)),
        pl.BlockSpec((D_IN, D_HID), lambda i: (0, 0)),
    ],
    out_specs=[
        pl.BlockSpec((BLK, D_OUT), lambda i: (i, 0)),
        pl.BlockSpec((BLK, D_OUT), lambda i: (i, 0)),
        pl.BlockSpec((BLK, 1), lambda i: (i, 0)),
    ],
    out_shape=[
        jax.ShapeDtypeStruct((NPAD, D_OUT), f32),
        jax.ShapeDtypeStruct((NPAD, D_OUT), f32),
        jax.ShapeDtypeStruct((NPAD, 1), f32),
    ],
)


def _mm2_body(sa_ref, sb_ref, u1a_ref, u1b_ref, dinv_ref, b1_ref, w2_ref,
              u2_ref):
    dv = dinv_ref[...]
    b1 = b1_ref[...]
    w2 = w2_ref[...]
    agg_a = sa_ref[0] + sa_ref[1] + u1a_ref[...]
    agg_b = sb_ref[0] + sb_ref[1] + u1b_ref[...]
    h_a = jnp.maximum(agg_a * dv + b1[:, :D_OUT], 0.0)
    h_b = jnp.maximum(agg_b * dv + b1[:, D_OUT:], 0.0)
    u2 = (jnp.dot(h_a, w2[:D_OUT], preferred_element_type=f32)
          + jnp.dot(h_b, w2[D_OUT:], preferred_element_type=f32))
    u2_ref[...] = u2 * dv


_mm2 = pl.pallas_call(
    _mm2_body,
    grid=(GRID,),
    in_specs=[
        pl.BlockSpec((NC, BLK, D_OUT), lambda i: (0, i, 0)),
        pl.BlockSpec((NC, BLK, D_OUT), lambda i: (0, i, 0)),
        pl.BlockSpec((BLK, D_OUT), lambda i: (i, 0)),
        pl.BlockSpec((BLK, D_OUT), lambda i: (i, 0)),
        pl.BlockSpec((BLK, 1), lambda i: (i, 0)),
        pl.BlockSpec((1, D_HID), lambda i: (0, 0)),
        pl.BlockSpec((D_HID, D_OUT), lambda i: (0, 0)),
    ],
    out_specs=pl.BlockSpec((BLK, D_OUT), lambda i: (i, 0)),
    out_shape=jax.ShapeDtypeStruct((NPAD, D_OUT), f32),
)


def _mm3_body(t_ref, u2_ref, dinv_ref, b2_ref, z_ref):
    z_ref[...] = ((t_ref[0] + t_ref[1] + u2_ref[...]) * dinv_ref[...]
                  + b2_ref[...])


_mm3 = pl.pallas_call(
    _mm3_body,
    grid=(GRID,),
    in_specs=[
        pl.BlockSpec((NC, BLK, D_OUT), lambda i: (0, i, 0)),
        pl.BlockSpec((BLK, D_OUT), lambda i: (i, 0)),
        pl.BlockSpec((BLK, 1), lambda i: (i, 0)),
        pl.BlockSpec((1, D_OUT), lambda i: (0, 0)),
    ],
    out_specs=pl.BlockSpec((BLK, D_OUT), lambda i: (i, 0)),
    out_shape=jax.ShapeDtypeStruct((NPAD, D_OUT), f32),
)


def _pad_edges(e):
    e = e.reshape(NT, EPT)
    pad = jnp.full((NT, EPT_PAD - EPT), N, dtype=i32)
    return jnp.concatenate([e, pad], axis=1).reshape(NT, NCHUNK, CH)


@jax.jit
def _run(x, edge_index, edge_label_index, W1, b1, W2, b2):
    src = _pad_edges(edge_index[0])
    dst = _pad_edges(edge_index[1])
    s_l = _pad_edges(edge_label_index[0])
    d_l = _pad_edges(edge_label_index[1])
    x_pad = jnp.pad(x, ((0, NPAD - N), (0, 0)))
    z64 = jnp.zeros((RPT, D_OUT), f32)

    degp = _sc_hist(dst)
    degp3 = degp.reshape(NC, NPAD, 1)
    u1a, u1b, dinv = _mm1(degp3, x_pad, W1)
    sa = _sc_scatter64(u1a, src, dst, z64)
    sb = _sc_scatter64(u1b, src, dst, z64)
    u2 = _mm2(sa, sb, u1a, u1b, dinv, b1.reshape(1, D_HID), W2)
    t_part = _sc_scatter64(u2, src, dst, z64)
    z = _mm3(t_part, u2, dinv, b2.reshape(1, D_OUT))
    sc = _sc_score(z, s_l, d_l)
    return sc.reshape(NT, EPT_PAD)[:, :EPT].reshape(E)


def kernel(x, edge_index, edge_label_index, W1, b1, W2, b2):
    return _run(x, edge_index, edge_label_index, W1, b1, W2, b2)


# speedup vs baseline: 1.1345x; 1.1345x over previous
import functools

import jax
import jax.numpy as jnp
from jax import lax
from jax.experimental import pallas as pl
from jax.experimental.pallas import tpu as pltpu
from jax.experimental.pallas import tpu_sc as plsc

N = 10000
E = 320000
D_IN = 128
D_HID = 128
D_OUT = 64
ATTRACT = 32

NC = 2
NS = 16
NT = NC * NS
CH = 128
EPT = E // NT
NCHUNK = (EPT + CH - 1) // CH
EPT_PAD = NCHUNK * CH
NPAD = 10240
RPT = NPAD // NS
BLK = 640
GRID = NPAD // BLK

_mesh = plsc.VectorSubcoreMesh(core_axis_name="c", subcore_axis_name="s")
f32 = jnp.float32
i32 = jnp.int32


@functools.partial(
    pl.kernel,
    out_type=jax.ShapeDtypeStruct((NC, NPAD), f32),
    mesh=_mesh,
    scratch_types=[
        pltpu.VMEM((NCHUNK, CH), i32),
        pltpu.VMEM((CH,), f32),
        pltpu.VMEM((RPT,), f32),
        pltpu.VMEM_SHARED((NPAD,), f32),
    ],
)
def _sc_hist(dst_hbm, out_hbm, dst_v, ones_v, zb_v, acc_sh):
    cid = lax.axis_index("c")
    sid = lax.axis_index("s")
    wid = cid * NS + sid
    pltpu.sync_copy(dst_hbm.at[wid], dst_v)
    for i in range(CH // 16):
        ones_v[pl.ds(i * 16, 16)] = jnp.ones((16,), f32)
    for i in range(RPT // 16):
        zb_v[pl.ds(i * 16, 16)] = jnp.zeros((16,), f32)
    pltpu.sync_copy(zb_v, acc_sh.at[pl.ds(sid * RPT, RPT)])
    plsc.subcore_barrier()

    def body(j, c):
        pltpu.sync_copy(ones_v, acc_sh.at[dst_v.at[j]], add=True)
        return c

    lax.fori_loop(0, NCHUNK, body, 0)
    plsc.subcore_barrier()
    pltpu.sync_copy(acc_sh.at[pl.ds(sid * RPT, RPT)],
                    out_hbm.at[cid, pl.ds(sid * RPT, RPT)])


D = D_OUT


@functools.partial(
    pl.kernel,
    out_type=jax.ShapeDtypeStruct((NC, NPAD, D), f32),
    mesh=_mesh,
    compiler_params=pltpu.CompilerParams(use_tc_tiling_on_sc=False),
    scratch_types=[
        pltpu.VMEM((NCHUNK, CH), i32),
        pltpu.VMEM((NCHUNK, CH), i32),
        pltpu.VMEM((CH, D), f32),
        pltpu.VMEM((CH, D), f32),
        pltpu.VMEM_SHARED((NPAD, D), f32),
        pltpu.VMEM_SHARED((NPAD, D), f32),
        pltpu.SemaphoreType.DMA,
        pltpu.SemaphoreType.DMA,
    ],
)
def _sc_scatter64(u_hbm, src_hbm, dst_hbm, zeros_hbm, out_hbm,
                  src_v, dst_v, r0, r1, acc_sh, u_sh, sem0, sem1):
    cid = lax.axis_index("c")
    sid = lax.axis_index("s")
    wid = cid * NS + sid
    pltpu.sync_copy(src_hbm.at[wid], src_v)
    pltpu.sync_copy(dst_hbm.at[wid], dst_v)
    pltpu.sync_copy(zeros_hbm, acc_sh.at[pl.ds(sid * RPT, RPT)])
    pltpu.sync_copy(u_hbm.at[pl.ds(sid * RPT, RPT)],
                    u_sh.at[pl.ds(sid * RPT, RPT)])
    plsc.subcore_barrier()

    def fire(j, buf, sem):
        pltpu.async_copy(u_sh.at[src_v.at[j]], buf, sem)

    def drain(j, buf, sem):
        pltpu.make_async_copy(u_sh.at[src_v.at[j]], buf, sem).wait()

    def scat(j, buf):
        pltpu.sync_copy(buf, acc_sh.at[dst_v.at[j]], add=True)

    fire(0, r0, sem0)

    def pair(k, c):
        j0 = 2 * k
        j1 = j0 + 1
        drain(j0, r0, sem0)
        fire(j1, r1, sem1)
        scat(j0, r0)
        drain(j1, r1, sem1)
        fire(j0 + 2, r0, sem0)
        scat(j1, r1)
        return c

    lax.fori_loop(0, (NCHUNK - 1) // 2, pair, 0)
    drain(NCHUNK - 1, r0, sem0)
    scat(NCHUNK - 1, r0)
    plsc.subcore_barrier()
    pltpu.sync_copy(acc_sh.at[pl.ds(sid * RPT, RPT)],
                    out_hbm.at[cid, pl.ds(sid * RPT, RPT)])


@functools.partial(
    pl.kernel,
    out_type=jax.ShapeDtypeStruct((NT, NCHUNK, CH), f32),
    mesh=_mesh,
    compiler_params=pltpu.CompilerParams(use_tc_tiling_on_sc=False,
                                         needs_layout_passes=False),
    scratch_types=[
        pltpu.VMEM((NCHUNK, CH), i32),
        pltpu.VMEM((NCHUNK, CH), i32),
        pltpu.VMEM((CH, D_OUT), f32),
        pltpu.VMEM((CH, D_OUT), f32),
        pltpu.VMEM((CH, D_OUT), f32),
        pltpu.VMEM((CH, D_OUT), f32),
        pltpu.VMEM((NCHUNK, CH), f32),
        pltpu.VMEM((16, 17), f32),
        pltpu.VMEM((16, 17), f32),
        pltpu.VMEM_SHARED((NPAD, D_OUT), f32),
        pltpu.SemaphoreType.DMA,
        pltpu.SemaphoreType.DMA,
    ],
)
def _sc_score(z_hbm, s_hbm, d_hbm, out_hbm,
              s_v, d_v, a0, b0, a1, b1, sc_v, tbuf0, tbuf1, z_sh, sem0, sem1):
    cid = lax.axis_index("c")
    sid = lax.axis_index("s")
    wid = cid * NS + sid
    pltpu.sync_copy(s_hbm.at[wid], s_v)
    pltpu.sync_copy(d_hbm.at[wid], d_v)
    pltpu.sync_copy(z_hbm.at[pl.ds(sid * RPT, RPT)],
                    z_sh.at[pl.ds(sid * RPT, RPT)])
    plsc.subcore_barrier()

    def fire(j, av, bv, sem):
        pltpu.async_copy(z_sh.at[s_v.at[j]], av, sem)
        pltpu.async_copy(z_sh.at[d_v.at[j]], bv, sem)

    def drain(j, av, bv, sem):
        pltpu.make_async_copy(z_sh.at[s_v.at[j]], av, sem).wait()
        pltpu.make_async_copy(z_sh.at[d_v.at[j]], bv, sem).wait()

    lane = lax.iota(i32, 16)

    def compute(j, av, bv):
        def phase1(g, tbuf):
            for t in range(16):
                e = g * 16 + t
                ra = (av[e, pl.ds(0, 16)] * bv[e, pl.ds(0, 16)]
                      + av[e, pl.ds(16, 16)] * bv[e, pl.ds(16, 16)])
                rb = (av[e, pl.ds(32, 16)] * bv[e, pl.ds(32, 16)]
                      + av[e, pl.ds(48, 16)] * bv[e, pl.ds(48, 16)])
                tbuf[t, pl.ds(0, 16)] = ra - rb

        def phase2(g, tbuf):
            acc0 = jnp.zeros((16,), f32)
            acc1 = jnp.zeros((16,), f32)
            for c in range(0, 16, 2):
                acc0 = acc0 + plsc.load_gather(tbuf, [lane, jnp.full((16,), c, i32)])
                acc1 = acc1 + plsc.load_gather(tbuf, [lane, jnp.full((16,), c + 1, i32)])
            sc_v[j, pl.ds(pl.multiple_of(g * 16, 16), 16)] = acc0 + acc1

        def gbody(g, c2):
            phase1(g, tbuf0)
            phase2(g, tbuf0)
            return c2

        lax.fori_loop(0, CH // 16, gbody, 0)

    fire(0, a0, b0, sem0)

    def pair(k, c):
        j0 = 2 * k
        j1 = j0 + 1
        drain(j0, a0, b0, sem0)
        fire(j1, a1, b1, sem1)
        compute(j0, a0, b0)
        drain(j1, a1, b1, sem1)
        fire(j0 + 2, a0, b0, sem0)
        compute(j1, a1, b1)
        return c

    lax.fori_loop(0, (NCHUNK - 1) // 2, pair, 0)
    drain(NCHUNK - 1, a0, b0, sem0)
    compute(NCHUNK - 1, a0, b0)
    pltpu.sync_copy(sc_v, out_hbm.at[wid])


def _mm1_body(degp_ref, x_ref, w1_ref, u1a_ref, u1b_ref, dinv_ref):
    deg = degp_ref[0] + degp_ref[1] + 1.0
    dv = lax.rsqrt(deg)
    dinv_ref[...] = dv
    xw = jnp.dot(x_ref[...], w1_ref[...], preferred_element_type=f32)
    u1a_ref[...] = xw[:, :D_OUT] * dv
    u1b_ref[...] = xw[:, D_OUT:] * dv


_mm1 = pl.pallas_call(
    _mm1_body,
    grid=(GRID,),
    in_specs=[
        pl.BlockSpec((NC, BLK, 1), lambda i: (0, i, 0)),
        pl.BlockSpec((BLK, D_IN), lambda i: (i, 0)),
        pl.BlockSpec((D_IN, D_HID), lambda i: (0, 0)),
    ],
    out_specs=[
        pl.BlockSpec((BLK, D_OUT), lambda i: (i, 0)),
        pl.BlockSpec((BLK, D_OUT), lambda i: (i, 0)),
        pl.BlockSpec((BLK, 1), lambda i: (i, 0)),
    ],
    out_shape=[
        jax.ShapeDtypeStruct((NPAD, D_OUT), f32),
        jax.ShapeDtypeStruct((NPAD, D_OUT), f32),
        jax.ShapeDtypeStruct((NPAD, 1), f32),
    ],
)


def _mm2_body(sa_ref, sb_ref, u1a_ref, u1b_ref, dinv_ref, b1_ref, w2_ref,
              u2_ref):
    dv = dinv_ref[...]
    b1 = b1_ref[...]
    w2 = w2_ref[...]
    agg_a = sa_ref[0] + sa_ref[1] + u1a_ref[...]
    agg_b = sb_ref[0] + sb_ref[1] + u1b_ref[...]
    h_a = jnp.maximum(agg_a * dv + b1[:, :D_OUT], 0.0)
    h_b = jnp.maximum(agg_b * dv + b1[:, D_OUT:], 0.0)
    u2 = (jnp.dot(h_a, w2[:D_OUT], preferred_element_type=f32)
          + jnp.dot(h_b, w2[D_OUT:], preferred_element_type=f32))
    u2_ref[...] = u2 * dv


_mm2 = pl.pallas_call(
    _mm2_body,
    grid=(GRID,),
    in_specs=[
        pl.BlockSpec((NC, BLK, D_OUT), lambda i: (0, i, 0)),
        pl.BlockSpec((NC, BLK, D_OUT), lambda i: (0, i, 0)),
        pl.BlockSpec((BLK, D_OUT), lambda i: (i, 0)),
        pl.BlockSpec((BLK, D_OUT), lambda i: (i, 0)),
        pl.BlockSpec((BLK, 1), lambda i: (i, 0)),
        pl.BlockSpec((1, D_HID), lambda i: (0, 0)),
        pl.BlockSpec((D_HID, D_OUT), lambda i: (0, 0)),
    ],
    out_specs=pl.BlockSpec((BLK, D_OUT), lambda i: (i, 0)),
    out_shape=jax.ShapeDtypeStruct((NPAD, D_OUT), f32),
)


def _mm3_body(t_ref, u2_ref, dinv_ref, b2_ref, z_ref):
    z_ref[...] = ((t_ref[0] + t_ref[1] + u2_ref[...]) * dinv_ref[...]
                  + b2_ref[...])


_mm3 = pl.pallas_call(
    _mm3_body,
    grid=(GRID,),
    in_specs=[
        pl.BlockSpec((NC, BLK, D_OUT), lambda i: (0, i, 0)),
        pl.BlockSpec((BLK, D_OUT), lambda i: (i, 0)),
        pl.BlockSpec((BLK, 1), lambda i: (i, 0)),
        pl.BlockSpec((1, D_OUT), lambda i: (0, 0)),
    ],
    out_specs=pl.BlockSpec((BLK, D_OUT), lambda i: (i, 0)),
    out_shape=jax.ShapeDtypeStruct((NPAD, D_OUT), f32),
)


def _pad_edges(e):
    e = e.reshape(NT, EPT)
    pad = jnp.full((NT, EPT_PAD - EPT), N, dtype=i32)
    return jnp.concatenate([e, pad], axis=1).reshape(NT, NCHUNK, CH)


@jax.jit
def _run(x, edge_index, edge_label_index, W1, b1, W2, b2):
    src = _pad_edges(edge_index[0])
    dst = _pad_edges(edge_index[1])
    s_l = _pad_edges(edge_label_index[0])
    d_l = _pad_edges(edge_label_index[1])
    x_pad = jnp.pad(x, ((0, NPAD - N), (0, 0)))
    z64 = jnp.zeros((RPT, D_OUT), f32)

    degp = _sc_hist(dst)
    degp3 = degp.reshape(NC, NPAD, 1)
    u1a, u1b, dinv = _mm1(degp3, x_pad, W1)
    sa = _sc_scatter64(u1a, src, dst, z64)
    sb = _sc_scatter64(u1b, src, dst, z64)
    u2 = _mm2(sa, sb, u1a, u1b, dinv, b1.reshape(1, D_HID), W2)
    t_part = _sc_scatter64(u2, src, dst, z64)
    z = _mm3(t_part, u2, dinv, b2.reshape(1, D_OUT))
    sc = _sc_score(z, s_l, d_l)
    return sc.reshape(NT, EPT_PAD)[:, :EPT].reshape(E)


def kernel(x, edge_index, edge_label_index, W1, b1, W2, b2):
    return _run(x, edge_index, edge_label_index, W1, b1, W2, b2)


# speedup vs baseline: 1.1371x; 1.0023x over previous
import functools

import jax
import jax.numpy as jnp
from jax import lax
from jax.experimental import pallas as pl
from jax.experimental.pallas import tpu as pltpu
from jax.experimental.pallas import tpu_sc as plsc

N = 10000
E = 320000
D_IN = 128
D_HID = 128
D_OUT = 64
ATTRACT = 32

NC = 2
NS = 16
NT = NC * NS
CH = 128
EPT = E // NT
NCHUNK = (EPT + CH - 1) // CH
EPT_PAD = NCHUNK * CH
NPAD = 10240
RPT = NPAD // NS
BLK = 640
GRID = NPAD // BLK

_mesh = plsc.VectorSubcoreMesh(core_axis_name="c", subcore_axis_name="s",
                               num_cores=NC, num_subcores=NS)
f32 = jnp.float32
i32 = jnp.int32


@functools.partial(
    pl.kernel,
    out_type=jax.ShapeDtypeStruct((NC, NPAD), f32),
    mesh=_mesh,
    scratch_types=[
        pltpu.VMEM((NCHUNK, CH), i32),
        pltpu.VMEM((CH,), f32),
        pltpu.VMEM((RPT,), f32),
        pltpu.VMEM_SHARED((NPAD,), f32),
    ],
)
def _sc_hist(dst_hbm, out_hbm, dst_v, ones_v, zb_v, acc_sh):
    cid = lax.axis_index("c")
    sid = lax.axis_index("s")
    wid = cid * NS + sid
    pltpu.sync_copy(dst_hbm.at[wid], dst_v)
    for i in range(CH // 16):
        ones_v[pl.ds(i * 16, 16)] = jnp.ones((16,), f32)
    for i in range(RPT // 16):
        zb_v[pl.ds(i * 16, 16)] = jnp.zeros((16,), f32)
    pltpu.sync_copy(zb_v, acc_sh.at[pl.ds(sid * RPT, RPT)])
    plsc.subcore_barrier()

    def body(j, c):
        pltpu.sync_copy(ones_v, acc_sh.at[dst_v.at[j]], add=True)
        return c

    lax.fori_loop(0, NCHUNK, body, 0)
    plsc.subcore_barrier()
    pltpu.sync_copy(acc_sh.at[pl.ds(sid * RPT, RPT)],
                    out_hbm.at[cid, pl.ds(sid * RPT, RPT)])


D = D_OUT


@functools.partial(
    pl.kernel,
    out_type=jax.ShapeDtypeStruct((NC, NPAD, D), f32),
    mesh=_mesh,
    compiler_params=pltpu.CompilerParams(use_tc_tiling_on_sc=False),
    scratch_types=[
        pltpu.VMEM((NCHUNK, CH), i32),
        pltpu.VMEM((NCHUNK, CH), i32),
        pltpu.VMEM((CH, D), f32),
        pltpu.VMEM((CH, D), f32),
        pltpu.VMEM_SHARED((NPAD, D), f32),
        pltpu.VMEM_SHARED((NPAD, D), f32),
        pltpu.SemaphoreType.DMA,
        pltpu.SemaphoreType.DMA,
    ],
)
def _sc_scatter64(u_hbm, src_hbm, dst_hbm, zeros_hbm, out_hbm,
                  src_v, dst_v, r0, r1, acc_sh, u_sh, sem0, sem1):
    cid = lax.axis_index("c")
    sid = lax.axis_index("s")
    wid = cid * NS + sid
    pltpu.sync_copy(src_hbm.at[wid], src_v)
    pltpu.sync_copy(dst_hbm.at[wid], dst_v)
    pltpu.sync_copy(zeros_hbm, acc_sh.at[pl.ds(sid * RPT, RPT)])
    pltpu.sync_copy(u_hbm.at[pl.ds(sid * RPT, RPT)],
                    u_sh.at[pl.ds(sid * RPT, RPT)])
    plsc.subcore_barrier()

    def fire(j, buf, sem):
        pltpu.async_copy(u_sh.at[src_v.at[j]], buf, sem)

    def drain(j, buf, sem):
        pltpu.make_async_copy(u_sh.at[src_v.at[j]], buf, sem).wait()

    def scat(j, buf):
        pltpu.sync_copy(buf, acc_sh.at[dst_v.at[j]], add=True)

    fire(0, r0, sem0)

    def pair(k, c):
        j0 = 2 * k
        j1 = j0 + 1
        drain(j0, r0, sem0)
        fire(j1, r1, sem1)
        scat(j0, r0)
        drain(j1, r1, sem1)
        fire(j0 + 2, r0, sem0)
        scat(j1, r1)
        return c

    lax.fori_loop(0, (NCHUNK - 1) // 2, pair, 0)
    drain(NCHUNK - 1, r0, sem0)
    scat(NCHUNK - 1, r0)
    plsc.subcore_barrier()
    pltpu.sync_copy(acc_sh.at[pl.ds(sid * RPT, RPT)],
                    out_hbm.at[cid, pl.ds(sid * RPT, RPT)])


@functools.partial(
    pl.kernel,
    out_type=jax.ShapeDtypeStruct((NT, NCHUNK, CH), f32),
    mesh=_mesh,
    compiler_params=pltpu.CompilerParams(use_tc_tiling_on_sc=False,
                                         needs_layout_passes=False),
    scratch_types=[
        pltpu.VMEM((NCHUNK, CH), i32),
        pltpu.VMEM((NCHUNK, CH), i32),
        pltpu.VMEM((CH, D_OUT), f32),
        pltpu.VMEM((CH, D_OUT), f32),
        pltpu.VMEM((CH, D_OUT), f32),
        pltpu.VMEM((CH, D_OUT), f32),
        pltpu.VMEM((NCHUNK, CH), f32),
        pltpu.VMEM((16, 17), f32),
        pltpu.VMEM((16, 17), f32),
        pltpu.VMEM_SHARED((NPAD, D_OUT), f32),
        pltpu.SemaphoreType.DMA,
        pltpu.SemaphoreType.DMA,
    ],
)
def _sc_score(z_hbm, s_hbm, d_hbm, out_hbm,
              s_v, d_v, a0, b0, a1, b1, sc_v, tbuf0, tbuf1, z_sh, sem0, sem1):
    cid = lax.axis_index("c")
    sid = lax.axis_index("s")
    wid = cid * NS + sid
    pltpu.sync_copy(s_hbm.at[wid], s_v)
    pltpu.sync_copy(d_hbm.at[wid], d_v)
    pltpu.sync_copy(z_hbm.at[pl.ds(sid * RPT, RPT)],
                    z_sh.at[pl.ds(sid * RPT, RPT)])
    plsc.subcore_barrier()

    def fire(j, av, bv, sem):
        pltpu.async_copy(z_sh.at[s_v.at[j]], av, sem)
        pltpu.async_copy(z_sh.at[d_v.at[j]], bv, sem)

    def drain(j, av, bv, sem):
        pltpu.make_async_copy(z_sh.at[s_v.at[j]], av, sem).wait()
        pltpu.make_async_copy(z_sh.at[d_v.at[j]], bv, sem).wait()

    lane = lax.iota(i32, 16)

    def compute(j, av, bv):
        def phase1(g, tbuf):
            for t in range(16):
                e = g * 16 + t
                ra = (av[e, pl.ds(0, 16)] * bv[e, pl.ds(0, 16)]
                      + av[e, pl.ds(16, 16)] * bv[e, pl.ds(16, 16)])
                rb = (av[e, pl.ds(32, 16)] * bv[e, pl.ds(32, 16)]
                      + av[e, pl.ds(48, 16)] * bv[e, pl.ds(48, 16)])
                tbuf[t, pl.ds(0, 16)] = ra - rb

        def phase2(g, tbuf):
            acc0 = jnp.zeros((16,), f32)
            acc1 = jnp.zeros((16,), f32)
            for c in range(0, 16, 2):
                acc0 = acc0 + plsc.load_gather(tbuf, [lane, jnp.full((16,), c, i32)])
                acc1 = acc1 + plsc.load_gather(tbuf, [lane, jnp.full((16,), c + 1, i32)])
            sc_v[j, pl.ds(pl.multiple_of(g * 16, 16), 16)] = acc0 + acc1

        def gbody(g, c2):
            phase1(g, tbuf0)
            phase2(g, tbuf0)
            return c2

        lax.fori_loop(0, CH // 16, gbody, 0)

    fire(0, a0, b0, sem0)

    def pair(k, c):
        j0 = 2 * k
        j1 = j0 + 1
        drain(j0, a0, b0, sem0)
        fire(j1, a1, b1, sem1)
        compute(j0, a0, b0)
        drain(j1, a1, b1, sem1)
        fire(j0 + 2, a0, b0, sem0)
        compute(j1, a1, b1)
        return c

    lax.fori_loop(0, (NCHUNK - 1) // 2, pair, 0)
    drain(NCHUNK - 1, a0, b0, sem0)
    compute(NCHUNK - 1, a0, b0)
    pltpu.sync_copy(sc_v, out_hbm.at[wid])


def _mm1_body(degp_ref, x_ref, w1_ref, u1a_ref, u1b_ref, dinv_ref):
    deg = degp_ref[0] + degp_ref[1] + 1.0
    dv = lax.rsqrt(deg)
    dinv_ref[...] = dv
    xw = jnp.dot(x_ref[...], w1_ref[...], preferred_element_type=f32)
    u1a_ref[...] = xw[:, :D_OUT] * dv
    u1b_ref[...] = xw[:, D_OUT:] * dv


_mm1 = pl.pallas_call(
    _mm1_body,
    grid=(GRID,),
    in_specs=[
        pl.BlockSpec((NC, BLK, 1), lambda i: (0, i, 0)),
        pl.BlockSpec((BLK, D_IN), lambda i: (i, 0)),
        pl.BlockSpec((D_IN, D_HID), lambda i: (0, 0)),
    ],
    out_specs=[
        pl.BlockSpec((BLK, D_OUT), lambda i: (i, 0)),
        pl.BlockSpec((BLK, D_OUT), lambda i: (i, 0)),
        pl.BlockSpec((BLK, 1), lambda i: (i, 0)),
    ],
    out_shape=[
        jax.ShapeDtypeStruct((NPAD, D_OUT), f32),
        jax.ShapeDtypeStruct((NPAD, D_OUT), f32),
        jax.ShapeDtypeStruct((NPAD, 1), f32),
    ],
)


def _mm2_body(sa_ref, sb_ref, u1a_ref, u1b_ref, dinv_ref, b1_ref, w2_ref,
              u2_ref):
    dv = dinv_ref[...]
    b1 = b1_ref[...]
    w2 = w2_ref[...]
    agg_a = sa_ref[0] + sa_ref[1] + u1a_ref[...]
    agg_b = sb_ref[0] + sb_ref[1] + u1b_ref[...]
    h_a = jnp.maximum(agg_a * dv + b1[:, :D_OUT], 0.0)
    h_b = jnp.maximum(agg_b * dv + b1[:, D_OUT:], 0.0)
    u2 = (jnp.dot(h_a, w2[:D_OUT], preferred_element_type=f32)
          + jnp.dot(h_b, w2[D_OUT:], preferred_element_type=f32))
    u2_ref[...] = u2 * dv


_mm2 = pl.pallas_call(
    _mm2_body,
    grid=(GRID,),
    in_specs=[
        pl.BlockSpec((NC, BLK, D_OUT), lambda i: (0, i, 0)),
        pl.BlockSpec((NC, BLK, D_OUT), lambda i: (0, i, 0)),
        pl.BlockSpec((BLK, D_OUT), lambda i: (i, 0)),
        pl.BlockSpec((BLK, D_OUT), lambda i: (i, 0)),
        pl.BlockSpec((BLK, 1), lambda i: (i, 0)),
        pl.BlockSpec((1, D_HID), lambda i: (0, 0)),
        pl.BlockSpec((D_HID, D_OUT), lambda i: (0, 0)),
    ],
    out_specs=pl.BlockSpec((BLK, D_OUT), lambda i: (i, 0)),
    out_shape=jax.ShapeDtypeStruct((NPAD, D_OUT), f32),
)


def _mm3_body(t_ref, u2_ref, dinv_ref, b2_ref, z_ref):
    z_ref[...] = ((t_ref[0] + t_ref[1] + u2_ref[...]) * dinv_ref[...]
                  + b2_ref[...])


_mm3 = pl.pallas_call(
    _mm3_body,
    grid=(GRID,),
    in_specs=[
        pl.BlockSpec((NC, BLK, D_OUT), lambda i: (0, i, 0)),
        pl.BlockSpec((BLK, D_OUT), lambda i: (i, 0)),
        pl.BlockSpec((BLK, 1), lambda i: (i, 0)),
        pl.BlockSpec((1, D_OUT), lambda i: (0, 0)),
    ],
    out_specs=pl.BlockSpec((BLK, D_OUT), lambda i: (i, 0)),
    out_shape=jax.ShapeDtypeStruct((NPAD, D_OUT), f32),
)


def _pad_edges(e):
    e = e.reshape(NT, EPT)
    pad = jnp.full((NT, EPT_PAD - EPT), N, dtype=i32)
    return jnp.concatenate([e, pad], axis=1).reshape(NT, NCHUNK, CH)


@jax.jit
def _run(x, edge_index, edge_label_index, W1, b1, W2, b2):
    src = _pad_edges(edge_index[0])
    dst = _pad_edges(edge_index[1])
    s_l = _pad_edges(edge_label_index[0])
    d_l = _pad_edges(edge_label_index[1])
    x_pad = jnp.pad(x, ((0, NPAD - N), (0, 0)))
    z64 = jnp.zeros((RPT, D_OUT), f32)

    degp = _sc_hist(dst)
    degp3 = degp.reshape(NC, NPAD, 1)
    u1a, u1b, dinv = _mm1(degp3, x_pad, W1)
    sa = _sc_scatter64(u1a, src, dst, z64)
    sb = _sc_scatter64(u1b, src, dst, z64)
    u2 = _mm2(sa, sb, u1a, u1b, dinv, b1.reshape(1, D_HID), W2)
    t_part = _sc_scatter64(u2, src, dst, z64)
    z = _mm3(t_part, u2, dinv, b2.reshape(1, D_OUT))
    sc = _sc_score(z, s_l, d_l)
    return sc.reshape(NT, EPT_PAD)[:, :EPT].reshape(E)


def kernel(x, edge_index, edge_label_index, W1, b1, W2, b2):
    return _run(x, edge_index, edge_label_index, W1, b1, W2, b2)
